# baseline (speedup 1.0000x reference)
import numpy as np
import jax
import jax.numpy as jnp
from jax.experimental import pallas as pl
from jax.experimental.pallas import tpu as pltpu

_B, _L, _C = 1024, 6, 2048
_BS = 32
_G = _B // _BS


def _build_consts():
    cpu = jax.local_devices(backend="cpu")[0]
    with jax.default_device(cpu):
        k = jax.random.key(1)
        k_table, k_noise = jax.random.split(k)
        rt = np.asarray(jax.random.uniform(k_table, (_B, 16), dtype=jnp.float32))
        noise = np.stack(
            [np.asarray(jax.random.normal(jax.random.fold_in(k_noise, i),
                                          (_B, _C), dtype=jnp.float32))
             for i in range(_L)], axis=1)

    a0 = np.where(rt[:, 0] < 0.1, 3.0 * rt[:, 0], 0.0).astype(np.float32)
    s = np.ones((_B, _L), np.float32)
    s[rt[:, 1] < 0.1, 0:3] *= -1.0
    s[rt[:, 2] < 0.1, 3:5] *= -1.0
    s[rt[:, 3] < 0.1, 5] *= -1.0
    zmask = rt[:, 4:10] < 0.1
    zmask[:, 1] = False
    z = np.where(zmask, 0.0, 1.0).astype(np.float32)
    c = np.where(rt[:, 10:16] < 0.1, rt[:, 10:16] * 3.0, 0.0).astype(np.float32)
    sz = (s * z).astype(np.float32)
    cz = (c * z).astype(np.float32)

    p = np.concatenate([sz, a0[:, None]], axis=1).astype(np.float32)

    bs_idx, ch_idx = np.nonzero(cz != 0.0)
    order = np.lexsort((bs_idx, ch_idx, bs_idx // _BS))
    bs_idx, ch_idx = bs_idx[order], ch_idx[order]
    wp = (cz[bs_idx, ch_idx, None] * noise[bs_idx, ch_idx, :]).astype(np.float32)
    na = wp.shape[0]
    na_pad = -(-na // 8) * 8
    wp = np.pad(wp, ((0, na_pad - na), (0, 0)))
    lb = np.pad((bs_idx % _BS).astype(np.int32), (0, na_pad - na))
    key = (bs_idx // _BS) * _L + ch_idx
    offs = np.searchsorted(key, np.arange(_G * _L + 1)).astype(np.int32)
    return p, wp, lb, offs


_P, _WP, _LB, _OFFS = _build_consts()
_NA_PAD = _WP.shape[0]


def _body(offs_ref, lb_ref, p_ref, x_ref, wp_ref, o_ref, std_ref):
    g = pl.program_id(0)
    x = x_ref[...]
    p = p_ref[...]
    a = p[:, _L:_L + 1]
    row0 = jax.lax.broadcasted_iota(jnp.int32, (1, _L, 1), 1) == 0
    y = x + jnp.where(row0, (a * x[:, 4, :])[:, None, :], 0.0)
    mean = jnp.mean(y, axis=2, keepdims=True)
    var = jnp.sum((y - mean) ** 2, axis=2, keepdims=True) * (1.0 / (_C - 1))
    std = jnp.sqrt(var)
    std_ref[...] = std[:, :, 0]
    sz = p[:, 0:_L]
    o_ref[...] = sz[:, :, None] * y

    def _mk(i):
        def add_one(r, carry):
            b = lb_ref[r]
            sv = std_ref[pl.ds(b, 1), pl.ds(i, 1)]
            w = wp_ref[pl.ds(r, 1), :]
            o_ref[pl.ds(b, 1), i, :] += sv * w
            return carry
        return add_one

    for i in range(_L):
        jax.lax.fori_loop(offs_ref[g * _L + i], offs_ref[g * _L + i + 1],
                          _mk(i), 0)


def kernel(X):
    grid_spec = pltpu.PrefetchScalarGridSpec(
        num_scalar_prefetch=2,
        grid=(_G,),
        in_specs=[
            pl.BlockSpec((_BS, _L + 1), lambda g, *_: (g, 0)),
            pl.BlockSpec((_BS, _L, _C), lambda g, *_: (g, 0, 0)),
            pl.BlockSpec((_NA_PAD, _C), lambda g, *_: (0, 0)),
        ],
        out_specs=pl.BlockSpec((_BS, _L, _C), lambda g, *_: (g, 0, 0)),
        scratch_shapes=[pltpu.VMEM((_BS, _L), jnp.float32)],
    )
    return pl.pallas_call(
        _body,
        grid_spec=grid_spec,
        out_shape=jax.ShapeDtypeStruct((_B, _L, _C), jnp.float32),
    )(jnp.asarray(_OFFS), jnp.asarray(_LB),
      jnp.asarray(_P), X, jnp.asarray(_WP))


# speedup vs baseline: 1.1694x; 1.1694x over previous
import numpy as np
import jax
import jax.numpy as jnp
from jax.experimental import pallas as pl

_B, _L, _C = 1024, 6, 2048
_BS = 128
_G = _B // _BS


def _build_consts():
    cpu = jax.local_devices(backend="cpu")[0]
    with jax.default_device(cpu):
        k = jax.random.key(1)
        k_table, k_noise = jax.random.split(k)
        rt = np.asarray(jax.random.uniform(k_table, (_B, 16), dtype=jnp.float32))
        noise = np.stack(
            [np.asarray(jax.random.normal(jax.random.fold_in(k_noise, i),
                                          (_B, _C), dtype=jnp.float32))
             for i in range(_L)], axis=1)

    a0 = np.where(rt[:, 0] < 0.1, 3.0 * rt[:, 0], 0.0).astype(np.float32)
    s = np.ones((_B, _L), np.float32)
    s[rt[:, 1] < 0.1, 0:3] *= -1.0
    s[rt[:, 2] < 0.1, 3:5] *= -1.0
    s[rt[:, 3] < 0.1, 5] *= -1.0
    zmask = rt[:, 4:10] < 0.1
    zmask[:, 1] = False
    z = np.where(zmask, 0.0, 1.0).astype(np.float32)
    c = np.where(rt[:, 10:16] < 0.1, rt[:, 10:16] * 3.0, 0.0).astype(np.float32)
    sz = (s * z).astype(np.float32)
    w = ((c * z)[:, :, None] * noise).astype(np.float32)
    p = np.concatenate([sz, a0[:, None]], axis=1).astype(np.float32)
    return p, w


_P, _W = _build_consts()


def _body(p_ref, x_ref, w_ref, o_ref):
    x = x_ref[...]
    p = p_ref[...]
    a = p[:, _L:_L + 1]
    y0 = x[:, 0, :] + a * x[:, 4, :]
    y = jnp.concatenate([y0[:, None, :], x[:, 1:, :]], axis=1)
    s1 = jnp.sum(y, axis=2, keepdims=True)
    s2 = jnp.sum(y * y, axis=2, keepdims=True)
    var = (s2 - s1 * s1 * (1.0 / _C)) * (1.0 / (_C - 1))
    std = jnp.sqrt(var)
    sz = p[:, 0:_L]
    o_ref[...] = sz[:, :, None] * y + std * w_ref[...]


def kernel(X):
    return pl.pallas_call(
        _body,
        out_shape=jax.ShapeDtypeStruct((_B, _L, _C), jnp.float32),
        grid=(_G,),
        in_specs=[
            pl.BlockSpec((_BS, _L + 1), lambda g: (g, 0)),
            pl.BlockSpec((_BS, _L, _C), lambda g: (g, 0, 0)),
            pl.BlockSpec((_BS, _L, _C), lambda g: (g, 0, 0)),
        ],
        out_specs=pl.BlockSpec((_BS, _L, _C), lambda g: (g, 0, 0)),
    )(jnp.asarray(_P), X, jnp.asarray(_W))
